# baseline (speedup 1.0000x reference)
import functools

import jax
import jax.numpy as jnp
from jax import lax
from jax.experimental import pallas as pl
from jax.experimental.pallas import tpu as pltpu
from jax.experimental.pallas import tpu_sc as plsc

N = 10000
E = 320000
D = 128
G = 64

NC = 2
NS = 16
NP = 10240
RW = NP // NS
EW = E // (NC * NS)
CH = 80
EWP = 10240
EPT = EWP // CH
SCC = 16
NST = EPT // SCC
BLK = 2000
TOT = float(N * D)

_mesh = plsc.VectorSubcoreMesh(
    core_axis_name="c", subcore_axis_name="s", num_cores=NC, num_subcores=NS)


@functools.partial(
    pl.kernel,
    out_type=jax.ShapeDtypeStruct((NC, NP), jnp.float32),
    mesh=_mesh,
    scratch_types=[
        pltpu.VMEM((EPT, CH), jnp.int32),
        pltpu.VMEM((CH,), jnp.float32),
        pltpu.VMEM((RW,), jnp.float32),
        pltpu.VMEM_SHARED((NP,), jnp.float32),
        pltpu.SemaphoreType.DMA,
    ],
)
def _sc_deg(dst_hbm, out_hbm, idx_v, ones_v, zero_v, deg_sh, sem):
    cid = lax.axis_index("c")
    sid = lax.axis_index("s")
    wid = cid * NS + sid

    for j in range(CH // 16):
        ones_v[pl.ds(j * 16, 16)] = jnp.full((16,), 1.0, jnp.float32)

    def zfill(i, carry):
        zero_v[pl.ds(i * 16, 16)] = jnp.zeros((16,), jnp.float32)
        return carry
    lax.fori_loop(0, RW // 16, zfill, 0)

    pltpu.sync_copy(zero_v, deg_sh.at[pl.ds(sid * RW, RW)])
    pltpu.sync_copy(dst_hbm.at[wid], idx_v)
    plsc.subcore_barrier()

    def body(k, carry):
        for t in range(4):
            pltpu.async_copy(ones_v, deg_sh.at[idx_v.at[k * 4 + t]], sem,
                             add=True)
        for t in range(4):
            pltpu.make_async_copy(
                ones_v, deg_sh.at[idx_v.at[k * 4 + t]], sem).wait()
        return carry
    lax.fori_loop(0, EPT // 4, body, 0)

    plsc.subcore_barrier()
    pltpu.sync_copy(deg_sh.at[pl.ds(sid * RW, RW)],
                    out_hbm.at[cid, pl.ds(sid * RW, RW)])


@functools.partial(
    pl.kernel,
    out_type=jax.ShapeDtypeStruct((NC, NP, D), jnp.float32),
    mesh=_mesh,
    scratch_types=[
        pltpu.VMEM((SCC, CH), jnp.int32),
        pltpu.VMEM((SCC, CH), jnp.int32),
        pltpu.VMEM((CH, D), jnp.float32),
        pltpu.VMEM((CH, D), jnp.float32),
        pltpu.VMEM((CH, D), jnp.float32),
        pltpu.VMEM((CH, D), jnp.float32),
        pltpu.VMEM_SHARED((NP, D), jnp.float32),
        pltpu.SemaphoreType.DMA,
        pltpu.SemaphoreType.DMA,
        pltpu.SemaphoreType.DMA,
        pltpu.SemaphoreType.DMA,
        pltpu.SemaphoreType.DMA,
        pltpu.SemaphoreType.DMA,
        pltpu.SemaphoreType.DMA,
        pltpu.SemaphoreType.DMA,
    ],
)
def _sc_pass(y_hbm, src_hbm, dst_hbm, out_hbm,
             src_v, dst_v, r0, r1, r2, r3, acc_sh,
             sg0, sg1, sg2, sg3, ss0, ss1, ss2, ss3):
    cid = lax.axis_index("c")
    sid = lax.axis_index("s")
    wid = cid * NS + sid
    rows = [r0, r1, r2, r3]
    sg = [sg0, sg1, sg2, sg3]
    ss = [ss0, ss1, ss2, ss3]

    def zfill(i, carry):
        for j in range(D // 16):
            r0[i, pl.ds(j * 16, 16)] = jnp.zeros((16,), jnp.float32)
        return carry
    lax.fori_loop(0, CH, zfill, 0)

    def zcopy(k, carry):
        pltpu.sync_copy(r0, acc_sh.at[pl.ds(sid * RW + k * CH, CH)])
        return carry
    lax.fori_loop(0, RW // CH, zcopy, 0)
    plsc.subcore_barrier()

    def g_start(j, b):
        pltpu.async_copy(y_hbm.at[src_v.at[j]], rows[b], sg[b])

    def g_wait(j, b):
        pltpu.make_async_copy(y_hbm.at[src_v.at[j]], rows[b], sg[b]).wait()

    def s_start(j, b):
        pltpu.async_copy(rows[b], acc_sh.at[dst_v.at[j]], ss[b], add=True)

    def s_wait(j, b):
        pltpu.make_async_copy(rows[b], acc_sh.at[dst_v.at[j]], ss[b]).wait()

    NQ = SCC // 4

    def stage(st, carry):
        pltpu.sync_copy(src_hbm.at[wid, pl.ds(st * SCC, SCC)], src_v)
        pltpu.sync_copy(dst_hbm.at[wid, pl.ds(st * SCC, SCC)], dst_v)
        g_start(0, 0)
        g_start(1, 1)
        g_start(2, 2)

        def quad(q, carry2):
            for t in range(4):
                j = 4 * q + t
                bp = (t + 3) % 4
                g_wait(j, t)
                s_start(j, t)
                if t == 0:
                    @pl.when(q > 0)
                    def _():
                        s_wait(j - 1, bp)
                    g_start(j + 3, bp)
                else:
                    s_wait(j - 1, bp)

                    @pl.when(q < NQ - 1)
                    def _():
                        g_start(j + 3, bp)
            return carry2
        lax.fori_loop(0, NQ, quad, 0)
        s_wait(SCC - 1, 3)
        return carry
    lax.fori_loop(0, NST, stage, 0)

    plsc.subcore_barrier()
    pltpu.sync_copy(acc_sh.at[pl.ds(sid * RW, RW)],
                    out_hbm.at[cid, pl.ds(sid * RW, RW)])


def _tc_dinv(deg2):
    def body(deg_ref, out_ref):
        out_ref[...] = lax.rsqrt(deg_ref[0:1, :] + deg_ref[1:2, :] + 1.0)
    return pl.pallas_call(
        body,
        out_shape=jax.ShapeDtypeStruct((1, NP), jnp.float32),
    )(deg2)


def _tc_pre(x, W, dinv_col):
    def body(x_ref, w_ref, d_ref, y_ref):
        y_ref[...] = d_ref[...] * jnp.dot(
            x_ref[...], w_ref[...], preferred_element_type=jnp.float32)
    return pl.pallas_call(
        body,
        grid=(N // BLK,),
        in_specs=[
            pl.BlockSpec((BLK, D), lambda i: (i, 0)),
            pl.BlockSpec((D, D), lambda i: (0, 0)),
            pl.BlockSpec((BLK, 1), lambda i: (i, 0)),
        ],
        out_specs=pl.BlockSpec((BLK, D), lambda i: (i, 0)),
        out_shape=jax.ShapeDtypeStruct((N, D), jnp.float32),
    )(x, W, dinv_col)


def _tc_stats(acc2, y, dinv_col, brow):
    def body(acc_ref, y_ref, d_ref, b_ref, pre_ref, st_ref, s_scr):
        i = pl.program_id(0)

        @pl.when(i == 0)
        def _():
            s_scr[0] = 0.0
            s_scr[1] = 0.0

        pre = d_ref[...] * (acc_ref[0] + acc_ref[1] + y_ref[...]) + b_ref[...]
        pre_ref[...] = pre
        s_scr[0] += jnp.sum(pre)
        s_scr[1] += jnp.sum(pre * pre)
        st_ref[0] = s_scr[0]
        st_ref[1] = s_scr[1]

    return pl.pallas_call(
        body,
        grid=(N // BLK,),
        in_specs=[
            pl.BlockSpec((NC, BLK, D), lambda i: (0, i, 0)),
            pl.BlockSpec((BLK, D), lambda i: (i, 0)),
            pl.BlockSpec((BLK, 1), lambda i: (i, 0)),
            pl.BlockSpec((1, D), lambda i: (0, 0)),
        ],
        out_specs=[
            pl.BlockSpec((BLK, D), lambda i: (i, 0)),
            pl.BlockSpec(memory_space=pltpu.SMEM),
        ],
        out_shape=[
            jax.ShapeDtypeStruct((N, D), jnp.float32),
            jax.ShapeDtypeStruct((2,), jnp.float32),
        ],
        scratch_shapes=[pltpu.SMEM((2,), jnp.float32)],
    )(acc2, y, dinv_col, brow)


def _normed(pre_ref, st_ref, w_ref, b_ref):
    mu = st_ref[0] / TOT
    var = st_ref[1] / TOT - mu * mu
    istd = lax.rsqrt(var + 1e-5)
    h = w_ref[...] * ((pre_ref[...] - mu) * istd) + b_ref[...]
    return jnp.where(h >= 0, h, 0.01 * h)


def _tc_layer(pre, st, lnw, lnb, W2, dinv_col):
    def body(pre_ref, st_ref, w_ref, b_ref, w2_ref, d_ref, y_ref):
        h = _normed(pre_ref, st_ref, w_ref, b_ref)
        y_ref[...] = d_ref[...] * jnp.dot(
            h, w2_ref[...], preferred_element_type=jnp.float32)
    return pl.pallas_call(
        body,
        grid=(N // BLK,),
        in_specs=[
            pl.BlockSpec((BLK, D), lambda i: (i, 0)),
            pl.BlockSpec(memory_space=pltpu.SMEM),
            pl.BlockSpec((1, D), lambda i: (0, 0)),
            pl.BlockSpec((1, D), lambda i: (0, 0)),
            pl.BlockSpec((D, D), lambda i: (0, 0)),
            pl.BlockSpec((BLK, 1), lambda i: (i, 0)),
        ],
        out_specs=pl.BlockSpec((BLK, D), lambda i: (i, 0)),
        out_shape=jax.ShapeDtypeStruct((N, D), jnp.float32),
    )(pre, st, lnw, lnb, W2, dinv_col)


def _tc_pool(pre, st, lnw, lnb, batch_col):
    def body(pre_ref, st_ref, w_ref, b_ref, bat_ref, out_ref, s_scr, c_scr):
        i = pl.program_id(0)

        @pl.when(i == 0)
        def _():
            s_scr[...] = jnp.zeros((G, D), jnp.float32)
            c_scr[...] = jnp.zeros((G, D), jnp.float32)

        h = _normed(pre_ref, st_ref, w_ref, b_ref)
        oh = (bat_ref[...] == lax.broadcasted_iota(
            jnp.int32, (BLK, G), 1)).astype(jnp.float32)
        dn = (((0,), (0,)), ((), ()))
        s_scr[...] += lax.dot_general(
            oh, h, dn, preferred_element_type=jnp.float32)
        c_scr[...] += lax.dot_general(
            oh, jnp.ones((BLK, D), jnp.float32), dn,
            preferred_element_type=jnp.float32)

        @pl.when(i == N // BLK - 1)
        def _():
            out_ref[...] = s_scr[...] / jnp.clip(c_scr[...], 1.0, None)

    return pl.pallas_call(
        body,
        grid=(N // BLK,),
        in_specs=[
            pl.BlockSpec((BLK, D), lambda i: (i, 0)),
            pl.BlockSpec(memory_space=pltpu.SMEM),
            pl.BlockSpec((1, D), lambda i: (0, 0)),
            pl.BlockSpec((1, D), lambda i: (0, 0)),
            pl.BlockSpec((BLK, 1), lambda i: (i, 0)),
        ],
        out_specs=pl.BlockSpec((G, D), lambda i: (0, 0)),
        out_shape=jax.ShapeDtypeStruct((G, D), jnp.float32),
        scratch_shapes=[
            pltpu.VMEM((G, D), jnp.float32),
            pltpu.VMEM((G, D), jnp.float32),
        ],
    )(pre, st, lnw, lnb, batch_col)


def kernel(x, edge_index, batch, W1, b1, ln1_w, ln1_b, W2, b2, ln2_w, ln2_b):
    nt = NC * NS
    pad = EWP - EW
    src = jnp.pad(edge_index[0].reshape(nt, EW), ((0, 0), (0, pad)),
                  constant_values=0).reshape(nt, EPT, CH)
    dst = jnp.pad(edge_index[1].reshape(nt, EW), ((0, 0), (0, pad)),
                  constant_values=NP - 1).reshape(nt, EPT, CH)

    deg2 = _sc_deg(dst)
    dinv_row = _tc_dinv(deg2)
    dinv_col = dinv_row[0, :N].reshape(N, 1)

    y1 = _tc_pre(x, W1, dinv_col)
    acc1 = _sc_pass(y1, src, dst)
    pre1, st1 = _tc_stats(acc1, y1, dinv_col, b1.reshape(1, D))
    y2 = _tc_layer(pre1, st1, ln1_w.reshape(1, D), ln1_b.reshape(1, D),
                   W2, dinv_col)
    acc2 = _sc_pass(y2, src, dst)
    pre2, st2 = _tc_stats(acc2, y2, dinv_col, b2.reshape(1, D))
    return _tc_pool(pre2, st2, ln2_w.reshape(1, D), ln2_b.reshape(1, D),
                    batch.reshape(N, 1))


# speedup vs baseline: 2.7187x; 2.7187x over previous
import functools

import jax
import jax.numpy as jnp
from jax import lax
from jax.experimental import pallas as pl
from jax.experimental.pallas import tpu as pltpu
from jax.experimental.pallas import tpu_sc as plsc

N = 10000
E = 320000
D = 128
G = 64

NC = 2
NS = 16
NP = 10240
RW = NP // NS
EW = E // (NC * NS)
CH = 80
EWP = 10240
EPT = EWP // CH
SCC = 16
NST = EPT // SCC
BLK = 2000
TOT = float(N * D)

_mesh = plsc.VectorSubcoreMesh(
    core_axis_name="c", subcore_axis_name="s", num_cores=NC, num_subcores=NS)


@functools.partial(
    pl.kernel,
    out_type=jax.ShapeDtypeStruct((NC, NP), jnp.float32),
    mesh=_mesh,
    scratch_types=[
        pltpu.VMEM((EPT, CH), jnp.int32),
        pltpu.VMEM((CH,), jnp.float32),
        pltpu.VMEM((RW,), jnp.float32),
        pltpu.VMEM_SHARED((NP,), jnp.float32),
        pltpu.SemaphoreType.DMA,
    ],
)
def _sc_deg(dst_hbm, out_hbm, idx_v, ones_v, zero_v, deg_sh, sem):
    cid = lax.axis_index("c")
    sid = lax.axis_index("s")
    wid = cid * NS + sid

    for j in range(CH // 16):
        ones_v[pl.ds(j * 16, 16)] = jnp.full((16,), 1.0, jnp.float32)

    def zfill(i, carry):
        zero_v[pl.ds(i * 16, 16)] = jnp.zeros((16,), jnp.float32)
        return carry
    lax.fori_loop(0, RW // 16, zfill, 0)

    pltpu.sync_copy(zero_v, deg_sh.at[pl.ds(sid * RW, RW)])
    pltpu.sync_copy(dst_hbm.at[wid], idx_v)
    plsc.subcore_barrier()

    def body(k, carry):
        for t in range(4):
            pltpu.async_copy(ones_v, deg_sh.at[idx_v.at[k * 4 + t]], sem,
                             add=True)
        for t in range(4):
            pltpu.make_async_copy(
                ones_v, deg_sh.at[idx_v.at[k * 4 + t]], sem).wait()
        return carry
    lax.fori_loop(0, EPT // 4, body, 0)

    plsc.subcore_barrier()
    pltpu.sync_copy(deg_sh.at[pl.ds(sid * RW, RW)],
                    out_hbm.at[cid, pl.ds(sid * RW, RW)])


@functools.partial(
    pl.kernel,
    out_type=jax.ShapeDtypeStruct((NC, NP, D), jnp.float32),
    mesh=_mesh,
    scratch_types=[
        pltpu.VMEM((SCC, CH), jnp.int32),
        pltpu.VMEM((SCC, CH), jnp.int32),
        pltpu.VMEM((CH, D), jnp.float32),
        pltpu.VMEM((CH, D), jnp.float32),
        pltpu.VMEM((CH, D), jnp.float32),
        pltpu.VMEM((CH, D), jnp.float32),
        pltpu.VMEM_SHARED((NP, D), jnp.float32),
        pltpu.SemaphoreType.DMA,
        pltpu.SemaphoreType.DMA,
        pltpu.SemaphoreType.DMA,
        pltpu.SemaphoreType.DMA,
        pltpu.SemaphoreType.DMA,
        pltpu.SemaphoreType.DMA,
        pltpu.SemaphoreType.DMA,
        pltpu.SemaphoreType.DMA,
    ],
)
def _sc_pass(y_hbm, src_hbm, dst_hbm, out_hbm,
             src_v, dst_v, r0, r1, r2, r3, acc_sh,
             sg0, sg1, sg2, sg3, ss0, ss1, ss2, ss3):
    cid = lax.axis_index("c")
    sid = lax.axis_index("s")
    wid = cid * NS + sid
    rows = [r0, r1, r2, r3]
    sg = [sg0, sg1, sg2, sg3]
    ss = [ss0, ss1, ss2, ss3]

    def zfill(i, carry):
        for j in range(D // 16):
            r0[i, pl.ds(j * 16, 16)] = jnp.zeros((16,), jnp.float32)
        return carry
    lax.fori_loop(0, CH, zfill, 0)

    def zcopy(k, carry):
        pltpu.sync_copy(r0, acc_sh.at[pl.ds(sid * RW + k * CH, CH)])
        return carry
    lax.fori_loop(0, RW // CH, zcopy, 0)
    plsc.subcore_barrier()

    def g_start(j, b):
        pltpu.async_copy(y_hbm.at[src_v.at[j]], rows[b], sg[b])

    def g_wait(j, b):
        pltpu.make_async_copy(y_hbm.at[src_v.at[j]], rows[b], sg[b]).wait()

    def s_start(j, b):
        pltpu.async_copy(rows[b], acc_sh.at[dst_v.at[j]], ss[b], add=True)

    def s_wait(j, b):
        pltpu.make_async_copy(rows[b], acc_sh.at[dst_v.at[j]], ss[b]).wait()

    NQ = SCC // 4

    def stage(st, carry):
        pltpu.sync_copy(src_hbm.at[wid, pl.ds(st * SCC, SCC)], src_v)
        pltpu.sync_copy(dst_hbm.at[wid, pl.ds(st * SCC, SCC)], dst_v)
        g_start(0, 0)
        g_start(1, 1)
        g_start(2, 2)

        def quad(q, carry2):
            for t in range(4):
                j = 4 * q + t
                bp = (t + 3) % 4
                g_wait(j, t)
                s_start(j, t)
                if t == 0:
                    @pl.when(q > 0)
                    def _():
                        s_wait(j - 1, bp)
                    g_start(j + 3, bp)
                else:
                    s_wait(j - 1, bp)

                    @pl.when(q < NQ - 1)
                    def _():
                        g_start(j + 3, bp)
            return carry2
        lax.fori_loop(0, NQ, quad, 0)
        s_wait(SCC - 1, 3)
        return carry
    lax.fori_loop(0, NST, stage, 0)

    plsc.subcore_barrier()
    pltpu.sync_copy(acc_sh.at[pl.ds(sid * RW, RW)],
                    out_hbm.at[cid, pl.ds(sid * RW, RW)])


def _tc_dinv(deg2):
    def body(deg_ref, out_ref):
        out_ref[...] = lax.rsqrt(deg_ref[0:1, :] + deg_ref[1:2, :] + 1.0)
    return pl.pallas_call(
        body,
        out_shape=jax.ShapeDtypeStruct((1, NP), jnp.float32),
    )(deg2)


def _tc_pre(x, W, dinv_col):
    def body(x_ref, w_ref, d_ref, y_ref):
        y_ref[...] = d_ref[...] * jnp.dot(
            x_ref[...], w_ref[...], preferred_element_type=jnp.float32)
    return pl.pallas_call(
        body,
        grid=(N // BLK,),
        in_specs=[
            pl.BlockSpec((BLK, D), lambda i: (i, 0)),
            pl.BlockSpec((D, D), lambda i: (0, 0)),
            pl.BlockSpec((BLK, 1), lambda i: (i, 0)),
        ],
        out_specs=pl.BlockSpec((BLK, D), lambda i: (i, 0)),
        out_shape=jax.ShapeDtypeStruct((N, D), jnp.float32),
    )(x, W, dinv_col)


def _tc_stats(acc2, y, dinv_col, brow):
    def body(acc_ref, y_ref, d_ref, b_ref, pre_ref, st_ref, s_scr):
        i = pl.program_id(0)

        @pl.when(i == 0)
        def _():
            s_scr[0] = 0.0
            s_scr[1] = 0.0

        pre = d_ref[...] * (acc_ref[0] + acc_ref[1] + y_ref[...]) + b_ref[...]
        pre_ref[...] = pre
        s_scr[0] += jnp.sum(pre)
        s_scr[1] += jnp.sum(pre * pre)
        st_ref[0] = s_scr[0]
        st_ref[1] = s_scr[1]

    return pl.pallas_call(
        body,
        grid=(N // BLK,),
        in_specs=[
            pl.BlockSpec((NC, BLK, D), lambda i: (0, i, 0)),
            pl.BlockSpec((BLK, D), lambda i: (i, 0)),
            pl.BlockSpec((BLK, 1), lambda i: (i, 0)),
            pl.BlockSpec((1, D), lambda i: (0, 0)),
        ],
        out_specs=[
            pl.BlockSpec((BLK, D), lambda i: (i, 0)),
            pl.BlockSpec(memory_space=pltpu.SMEM),
        ],
        out_shape=[
            jax.ShapeDtypeStruct((N, D), jnp.float32),
            jax.ShapeDtypeStruct((2,), jnp.float32),
        ],
        scratch_shapes=[pltpu.SMEM((2,), jnp.float32)],
    )(acc2, y, dinv_col, brow)


def _normed(pre_ref, st_ref, w_ref, b_ref):
    mu = st_ref[0] / TOT
    var = st_ref[1] / TOT - mu * mu
    istd = lax.rsqrt(var + 1e-5)
    h = w_ref[...] * ((pre_ref[...] - mu) * istd) + b_ref[...]
    return jnp.where(h >= 0, h, 0.01 * h)


def _tc_layer(pre, st, lnw, lnb, W2, dinv_col):
    def body(pre_ref, st_ref, w_ref, b_ref, w2_ref, d_ref, y_ref):
        h = _normed(pre_ref, st_ref, w_ref, b_ref)
        y_ref[...] = d_ref[...] * jnp.dot(
            h, w2_ref[...], preferred_element_type=jnp.float32)
    return pl.pallas_call(
        body,
        grid=(N // BLK,),
        in_specs=[
            pl.BlockSpec((BLK, D), lambda i: (i, 0)),
            pl.BlockSpec(memory_space=pltpu.SMEM),
            pl.BlockSpec((1, D), lambda i: (0, 0)),
            pl.BlockSpec((1, D), lambda i: (0, 0)),
            pl.BlockSpec((D, D), lambda i: (0, 0)),
            pl.BlockSpec((BLK, 1), lambda i: (i, 0)),
        ],
        out_specs=pl.BlockSpec((BLK, D), lambda i: (i, 0)),
        out_shape=jax.ShapeDtypeStruct((N, D), jnp.float32),
    )(pre, st, lnw, lnb, W2, dinv_col)


def _tc_pool(pre, st, lnw, lnb, batch_col):
    def body(pre_ref, st_ref, w_ref, b_ref, bat_ref, out_ref, s_scr, c_scr):
        i = pl.program_id(0)

        @pl.when(i == 0)
        def _():
            s_scr[...] = jnp.zeros((G, D), jnp.float32)
            c_scr[...] = jnp.zeros((G, D), jnp.float32)

        h = _normed(pre_ref, st_ref, w_ref, b_ref)
        oh = (bat_ref[...] == lax.broadcasted_iota(
            jnp.int32, (BLK, G), 1)).astype(jnp.float32)
        dn = (((0,), (0,)), ((), ()))
        s_scr[...] += lax.dot_general(
            oh, h, dn, preferred_element_type=jnp.float32)
        c_scr[...] += lax.dot_general(
            oh, jnp.ones((BLK, D), jnp.float32), dn,
            preferred_element_type=jnp.float32)

        @pl.when(i == N // BLK - 1)
        def _():
            out_ref[...] = s_scr[...] / jnp.clip(c_scr[...], 1.0, None)

    return pl.pallas_call(
        body,
        grid=(N // BLK,),
        in_specs=[
            pl.BlockSpec((BLK, D), lambda i: (i, 0)),
            pl.BlockSpec(memory_space=pltpu.SMEM),
            pl.BlockSpec((1, D), lambda i: (0, 0)),
            pl.BlockSpec((1, D), lambda i: (0, 0)),
            pl.BlockSpec((BLK, 1), lambda i: (i, 0)),
        ],
        out_specs=pl.BlockSpec((G, D), lambda i: (0, 0)),
        out_shape=jax.ShapeDtypeStruct((G, D), jnp.float32),
        scratch_shapes=[
            pltpu.VMEM((G, D), jnp.float32),
            pltpu.VMEM((G, D), jnp.float32),
        ],
    )(pre, st, lnw, lnb, batch_col)


def kernel(x, edge_index, batch, W1, b1, ln1_w, ln1_b, W2, b2, ln2_w, ln2_b):
    nt = NC * NS
    pad = EWP - EW
    src_pad = jnp.broadcast_to(jnp.arange(pad, dtype=jnp.int32), (nt, pad))
    dst_pad = jnp.broadcast_to(jnp.arange(pad, dtype=jnp.int32) + N,
                               (nt, pad))
    src = jnp.concatenate([edge_index[0].reshape(nt, EW), src_pad],
                          axis=1).reshape(nt, EPT, CH)
    dst = jnp.concatenate([edge_index[1].reshape(nt, EW), dst_pad],
                          axis=1).reshape(nt, EPT, CH)

    deg2 = _sc_deg(dst)
    dinv_row = _tc_dinv(deg2)
    dinv_col = dinv_row[0, :N].reshape(N, 1)

    y1 = _tc_pre(x, W1, dinv_col)
    acc1 = _sc_pass(y1, src, dst)
    pre1, st1 = _tc_stats(acc1, y1, dinv_col, b1.reshape(1, D))
    y2 = _tc_layer(pre1, st1, ln1_w.reshape(1, D), ln1_b.reshape(1, D),
                   W2, dinv_col)
    acc2 = _sc_pass(y2, src, dst)
    pre2, st2 = _tc_stats(acc2, y2, dinv_col, b2.reshape(1, D))
    return _tc_pool(pre2, st2, ln2_w.reshape(1, D), ln2_b.reshape(1, D),
                    batch.reshape(N, 1))
